# baseline (speedup 1.0000x reference)
import functools

import jax
import jax.numpy as jnp
from jax import lax
from jax.experimental import pallas as pl
from jax.experimental.pallas import tpu as pltpu
from jax.experimental.pallas import tpu_sc as plsc

_NC, _NS = 2, 16
_CHUNK = 1000


def _sc_gather_call(table, idx_flat):
    R = idx_flat.shape[0]
    W = table.shape[1]
    nw = _NC * _NS
    b_per_w = R // nw
    n_ch = b_per_w // _CHUNK
    mesh = plsc.VectorSubcoreMesh(core_axis_name="c", subcore_axis_name="s")

    @functools.partial(
        pl.kernel,
        mesh=mesh,
        out_type=jax.ShapeDtypeStruct((R, W), jnp.float32),
        scratch_types=[
            pltpu.VMEM((_CHUNK,), jnp.int32),
            pltpu.VMEM((_CHUNK, W), jnp.float32),
            pltpu.SemaphoreType.DMA,
        ],
    )
    def k(t_hbm, i_hbm, o_hbm, idx_v, rows_v, sem):
        wid = lax.axis_index("s") * _NC + lax.axis_index("c")
        base = wid * b_per_w

        @pl.loop(0, n_ch)
        def _(c):
            off = base + c * _CHUNK
            pltpu.sync_copy(i_hbm.at[pl.ds(off, _CHUNK)], idx_v)
            pltpu.async_copy(t_hbm.at[idx_v], rows_v, sem).wait()
            pltpu.sync_copy(rows_v, o_hbm.at[pl.ds(off, _CHUNK)])

    return k(table, idx_flat)


def _tc_body(g_ref, opts_ref, w1_ref, b1_ref, w2_ref, b2_ref,
             w3_ref, b3_ref, wm_ref, bias_ref, out_ref):
    K = g_ref.shape[0]
    KC = w3_ref.shape[1]
    C = wm_ref.shape[0] // KC
    opts = opts_ref[...]
    w1 = w1_ref[...]
    b1 = b1_ref[...]
    w2 = w2_ref[...]
    b2 = b2_ref[...]
    w3 = w3_ref[...]
    b3 = b3_ref[...]
    accs = [None] * KC
    for k in range(K):
        g = g_ref[k]
        rel = g[:, C:C + 16] - opts
        h1 = jax.nn.relu(jnp.dot(rel, w1, preferred_element_type=jnp.float32) + b1)
        h2 = jax.nn.relu(jnp.dot(h1, w2, preferred_element_type=jnp.float32) + b2)
        dk = jax.nn.relu(jnp.dot(h2, w3, preferred_element_type=jnp.float32) + b3)
        f = g[:, :C]
        for j in range(KC):
            term = f * dk[:, j:j + 1]
            accs[j] = term if k == 0 else accs[j] + term
    cat = jnp.concatenate(accs, axis=1)
    out = jnp.dot(cat, wm_ref[...], preferred_element_type=jnp.float32)
    out_ref[...] = out * (1.0 / K) + bias_ref[...]


def kernel(features, input_pts, neighbor_num, output_pts, normalize, indices_,
           weight, bias, centers, l1_W, l1_b, l2_W, l2_b, l3_W, l3_b):
    B, N, C = features.shape
    K = indices_.shape[2]
    D = input_pts.shape[2]
    KC = centers.shape[1]
    H1 = l1_W.shape[0]
    OUT = weight.shape[2]
    NB = 400

    feats2d = features.reshape(N, C)
    pts2d = input_pts.reshape(N, D)
    table = jnp.concatenate(
        [feats2d, pts2d, jnp.zeros((N, 128 - C - D), jnp.float32)], axis=1)
    opts16 = jnp.pad(output_pts.reshape(N, D), ((0, 0), (0, 16 - D)))
    idx_km = indices_.reshape(N, K).T.reshape(-1).astype(jnp.int32)

    g3 = _sc_gather_call(table, idx_km).reshape(K, N, 128)

    w1eff = l1_W.reshape(H1, D, KC).sum(-1).T
    w1p = jnp.pad(w1eff, ((0, 16 - D), (0, 0)))
    b1eff = (l1_b - (l1_W.reshape(H1, D, KC) * centers[None]).sum((1, 2)))
    wm = weight.transpose(1, 0, 2).reshape(KC * C, OUT)

    out = pl.pallas_call(
        _tc_body,
        grid=(N // NB,),
        in_specs=[
            pl.BlockSpec((K, NB, 128), lambda b: (0, b, 0)),
            pl.BlockSpec((NB, 16), lambda b: (b, 0)),
            pl.BlockSpec((16, H1), lambda b: (0, 0)),
            pl.BlockSpec((1, H1), lambda b: (0, 0)),
            pl.BlockSpec((H1, KC), lambda b: (0, 0)),
            pl.BlockSpec((1, KC), lambda b: (0, 0)),
            pl.BlockSpec((KC, KC), lambda b: (0, 0)),
            pl.BlockSpec((1, KC), lambda b: (0, 0)),
            pl.BlockSpec((KC * C, OUT), lambda b: (0, 0)),
            pl.BlockSpec((1, OUT), lambda b: (0, 0)),
        ],
        out_specs=pl.BlockSpec((NB, OUT), lambda b: (b, 0)),
        out_shape=jax.ShapeDtypeStruct((N, OUT), jnp.float32),
    )(g3, opts16, w1p, b1eff.reshape(1, H1), l2_W.T,
      l2_b.reshape(1, KC), l3_W.T, l3_b.reshape(1, KC), wm,
      bias.reshape(1, OUT))

    return (out.reshape(B, N, OUT), output_pts)


# speedup vs baseline: 2.5147x; 2.5147x over previous
import functools

import jax
import jax.numpy as jnp
from jax import lax
from jax.experimental import pallas as pl
from jax.experimental.pallas import tpu as pltpu
from jax.experimental.pallas import tpu_sc as plsc

_NC, _NS = 2, 16
_CHUNK = 1000


def _sc_gather_call(table, idx_flat):
    R = idx_flat.shape[0]
    W = table.shape[1]
    nw = _NC * _NS
    b_per_w = R // nw
    n_ch = b_per_w // _CHUNK
    mesh = plsc.VectorSubcoreMesh(core_axis_name="c", subcore_axis_name="s")

    @functools.partial(
        pl.kernel,
        mesh=mesh,
        out_type=jax.ShapeDtypeStruct((R, W), jnp.float32),
        scratch_types=[
            pltpu.VMEM((_CHUNK,), jnp.int32),
            pltpu.VMEM((_CHUNK, W), jnp.float32),
            pltpu.SemaphoreType.DMA,
        ],
    )
    def k(t_hbm, i_hbm, o_hbm, idx_v, rows_v, sem):
        wid = lax.axis_index("s") * _NC + lax.axis_index("c")
        base = wid * b_per_w

        @pl.loop(0, n_ch)
        def _(c):
            off = base + c * _CHUNK
            pltpu.sync_copy(i_hbm.at[pl.ds(off, _CHUNK)], idx_v)
            pltpu.async_copy(t_hbm.at[idx_v], rows_v, sem).wait()
            pltpu.sync_copy(rows_v, o_hbm.at[pl.ds(off, _CHUNK)])

    return k(table, idx_flat)


def _tc_body(g_ref, opts_ref, w1_ref, b1_ref, w2_ref, b2_ref,
             w3_ref, b3_ref, wm_ref, bias_ref, esp_ref, out_ref):
    K = g_ref.shape[0]
    NB = g_ref.shape[1]
    KC = w3_ref.shape[1]
    C = wm_ref.shape[0] // KC
    opts = opts_ref[...]
    g2 = g_ref[...].reshape(K * NB, 128)
    rel = g2[:, C:C + 16] - jnp.tile(opts, (K, 1))
    h1 = jax.nn.relu(jnp.dot(rel, w1_ref[...],
                             preferred_element_type=jnp.float32) + b1_ref[...])
    h2 = jax.nn.relu(jnp.dot(h1, w2_ref[...],
                             preferred_element_type=jnp.float32) + b2_ref[...])
    dall = jax.nn.relu(jnp.dot(h2, w3_ref[...],
                               preferred_element_type=jnp.float32) + b3_ref[...])
    d3 = dall.reshape(K, NB, KC)
    accs = [None] * KC
    for k in range(K):
        dsp = jnp.dot(d3[k], esp_ref[...], preferred_element_type=jnp.float32)
        f = g_ref[k][:, :C]
        for j in range(KC):
            term = f * dsp[:, j * C:(j + 1) * C]
            accs[j] = term if k == 0 else accs[j] + term
    cat = jnp.concatenate(accs, axis=1)
    out = jnp.dot(cat, wm_ref[...], preferred_element_type=jnp.float32)
    out_ref[...] = out * (1.0 / K) + bias_ref[...]


def kernel(features, input_pts, neighbor_num, output_pts, normalize, indices_,
           weight, bias, centers, l1_W, l1_b, l2_W, l2_b, l3_W, l3_b):
    B, N, C = features.shape
    K = indices_.shape[2]
    D = input_pts.shape[2]
    KC = centers.shape[1]
    H1 = l1_W.shape[0]
    OUT = weight.shape[2]
    NB = 400

    feats2d = features.reshape(N, C)
    pts2d = input_pts.reshape(N, D)
    table = jnp.concatenate(
        [feats2d, pts2d, jnp.zeros((N, 128 - C - D), jnp.float32)], axis=1)
    opts16 = jnp.pad(output_pts.reshape(N, D), ((0, 0), (0, 16 - D)))
    idx_km = indices_.reshape(N, K).T.reshape(-1).astype(jnp.int32)

    g3 = _sc_gather_call(table, idx_km).reshape(K, N, 128)

    w1eff = l1_W.reshape(H1, D, KC).sum(-1).T
    w1p = jnp.pad(w1eff, ((0, 16 - D), (0, 0)))
    b1eff = (l1_b - (l1_W.reshape(H1, D, KC) * centers[None]).sum((1, 2)))
    wm = weight.transpose(1, 0, 2).reshape(KC * C, OUT)
    esp = jnp.kron(jnp.eye(KC, dtype=jnp.float32),
                   jnp.ones((1, C), jnp.float32))

    out = pl.pallas_call(
        _tc_body,
        grid=(N // NB,),
        in_specs=[
            pl.BlockSpec((K, NB, 128), lambda b: (0, b, 0)),
            pl.BlockSpec((NB, 16), lambda b: (b, 0)),
            pl.BlockSpec((16, H1), lambda b: (0, 0)),
            pl.BlockSpec((1, H1), lambda b: (0, 0)),
            pl.BlockSpec((H1, KC), lambda b: (0, 0)),
            pl.BlockSpec((1, KC), lambda b: (0, 0)),
            pl.BlockSpec((KC, KC), lambda b: (0, 0)),
            pl.BlockSpec((1, KC), lambda b: (0, 0)),
            pl.BlockSpec((KC * C, OUT), lambda b: (0, 0)),
            pl.BlockSpec((1, OUT), lambda b: (0, 0)),
            pl.BlockSpec((KC, KC * C), lambda b: (0, 0)),
        ],
        out_specs=pl.BlockSpec((NB, OUT), lambda b: (b, 0)),
        out_shape=jax.ShapeDtypeStruct((N, OUT), jnp.float32),
    )(g3, opts16, w1p, b1eff.reshape(1, H1), l2_W.T,
      l2_b.reshape(1, KC), l3_W.T, l3_b.reshape(1, KC), wm,
      bias.reshape(1, OUT), esp)

    return (out.reshape(B, N, OUT), output_pts)


# speedup vs baseline: 4.2255x; 1.6803x over previous
import functools

import jax
import jax.numpy as jnp
from jax import lax
from jax.experimental import pallas as pl
from jax.experimental.pallas import tpu as pltpu
from jax.experimental.pallas import tpu_sc as plsc

_NC, _NS = 2, 16
_CHUNK = 1000


def _sc_gather_call(table, idx_flat):
    R = idx_flat.shape[0]
    W = table.shape[1]
    nw = _NC * _NS
    b_per_w = R // nw
    n_ch = b_per_w // _CHUNK
    mesh = plsc.VectorSubcoreMesh(core_axis_name="c", subcore_axis_name="s")

    @functools.partial(
        pl.kernel,
        mesh=mesh,
        out_type=jax.ShapeDtypeStruct((R, W), jnp.float32),
        scratch_types=[
            pltpu.VMEM((_CHUNK,), jnp.int32),
            pltpu.VMEM((_CHUNK, W), jnp.float32),
            pltpu.SemaphoreType.DMA,
        ],
    )
    def k(t_hbm, i_hbm, o_hbm, idx_v, rows_v, sem):
        wid = lax.axis_index("s") * _NC + lax.axis_index("c")
        base = wid * b_per_w

        @pl.loop(0, n_ch)
        def _(c):
            off = base + c * _CHUNK
            pltpu.sync_copy(i_hbm.at[pl.ds(off, _CHUNK)], idx_v)
            pltpu.async_copy(t_hbm.at[idx_v], rows_v, sem).wait()
            pltpu.sync_copy(rows_v, o_hbm.at[pl.ds(off, _CHUNK)])

    return k(table, idx_flat)


def _tc_body(g_ref, opts_ref, w1_ref, b1_ref, w2_ref, b2_ref,
             w3_ref, b3_ref, wm_ref, bias_ref, esp_ref, out_ref):
    K = g_ref.shape[0]
    NB = g_ref.shape[1]
    KC = w3_ref.shape[1]
    C = wm_ref.shape[0] // KC
    opts = opts_ref[...]
    g2 = g_ref[...].reshape(K * NB, 128)
    rel = g2[:, C:C + 16] - jnp.tile(opts, (K, 1))
    h1 = jax.nn.relu(jnp.dot(rel, w1_ref[...],
                             preferred_element_type=jnp.float32) + b1_ref[...])
    h2 = jax.nn.relu(jnp.dot(h1, w2_ref[...],
                             preferred_element_type=jnp.float32) + b2_ref[...])
    dall = jax.nn.relu(jnp.dot(h2, w3_ref[...],
                               preferred_element_type=jnp.float32) + b3_ref[...])
    d3 = dall.reshape(K, NB, KC)
    accs = [None] * (KC // 2)
    for k in range(K):
        dsp = jnp.dot(d3[k], esp_ref[...], preferred_element_type=jnp.float32)
        f = g_ref[k][:, :C]
        f2 = jnp.concatenate([f, f], axis=1)
        for j2 in range(KC // 2):
            term = f2 * dsp[:, j2 * 2 * C:(j2 + 1) * 2 * C]
            accs[j2] = term if k == 0 else accs[j2] + term
    cat = jnp.concatenate(accs, axis=1)
    out = jnp.dot(cat, wm_ref[...], preferred_element_type=jnp.float32)
    out_ref[...] = out * (1.0 / K) + bias_ref[...]


def kernel(features, input_pts, neighbor_num, output_pts, normalize, indices_,
           weight, bias, centers, l1_W, l1_b, l2_W, l2_b, l3_W, l3_b):
    B, N, C = features.shape
    K = indices_.shape[2]
    D = input_pts.shape[2]
    KC = centers.shape[1]
    H1 = l1_W.shape[0]
    OUT = weight.shape[2]
    NB = 400

    feats2d = features.reshape(N, C)
    pts2d = input_pts.reshape(N, D)
    table = jnp.concatenate(
        [feats2d, pts2d, jnp.zeros((N, 128 - C - D), jnp.float32)], axis=1)
    opts16 = jnp.pad(output_pts.reshape(N, D), ((0, 0), (0, 16 - D)))
    idx_km = indices_.reshape(N, K).T.reshape(-1).astype(jnp.int32)

    g3 = _sc_gather_call(table, idx_km).reshape(K, N, 128)

    w1eff = l1_W.reshape(H1, D, KC).sum(-1).T
    w1p = jnp.pad(w1eff, ((0, 16 - D), (0, 0)))
    b1eff = (l1_b - (l1_W.reshape(H1, D, KC) * centers[None]).sum((1, 2)))
    wm = weight.transpose(1, 0, 2).reshape(KC * C, OUT)
    esp = jnp.kron(jnp.eye(KC, dtype=jnp.float32),
                   jnp.ones((1, C), jnp.float32))

    out = pl.pallas_call(
        _tc_body,
        grid=(N // NB,),
        in_specs=[
            pl.BlockSpec((K, NB, 128), lambda b: (0, b, 0)),
            pl.BlockSpec((NB, 16), lambda b: (b, 0)),
            pl.BlockSpec((16, H1), lambda b: (0, 0)),
            pl.BlockSpec((1, H1), lambda b: (0, 0)),
            pl.BlockSpec((H1, KC), lambda b: (0, 0)),
            pl.BlockSpec((1, KC), lambda b: (0, 0)),
            pl.BlockSpec((KC, KC), lambda b: (0, 0)),
            pl.BlockSpec((1, KC), lambda b: (0, 0)),
            pl.BlockSpec((KC * C, OUT), lambda b: (0, 0)),
            pl.BlockSpec((1, OUT), lambda b: (0, 0)),
            pl.BlockSpec((KC, KC * C), lambda b: (0, 0)),
        ],
        out_specs=pl.BlockSpec((NB, OUT), lambda b: (b, 0)),
        out_shape=jax.ShapeDtypeStruct((N, OUT), jnp.float32),
    )(g3, opts16, w1p, b1eff.reshape(1, H1), l2_W.T,
      l2_b.reshape(1, KC), l3_W.T, l3_b.reshape(1, KC), wm,
      bias.reshape(1, OUT), esp)

    return (out.reshape(B, N, OUT), output_pts)


# speedup vs baseline: 5.1957x; 1.2296x over previous
import functools

import jax
import jax.numpy as jnp
from jax import lax
from jax.experimental import pallas as pl
from jax.experimental.pallas import tpu as pltpu
from jax.experimental.pallas import tpu_sc as plsc

_NC, _NS = 2, 16
_CHUNK = 1000


def _sc_gather_call(table, idx_flat):
    R = idx_flat.shape[0]
    W = table.shape[1]
    nw = _NC * _NS
    b_per_w = R // nw
    n_ch = b_per_w // _CHUNK
    mesh = plsc.VectorSubcoreMesh(core_axis_name="c", subcore_axis_name="s")

    @functools.partial(
        pl.kernel,
        mesh=mesh,
        out_type=jax.ShapeDtypeStruct((R, W), jnp.float32),
        scratch_types=[
            pltpu.VMEM((_CHUNK,), jnp.int32),
            pltpu.VMEM((_CHUNK, W), jnp.float32),
            pltpu.SemaphoreType.DMA,
        ],
    )
    def k(t_hbm, i_hbm, o_hbm, idx_v, rows_v, sem):
        wid = lax.axis_index("s") * _NC + lax.axis_index("c")
        base = wid * b_per_w

        @pl.loop(0, n_ch)
        def _(c):
            off = base + c * _CHUNK
            pltpu.sync_copy(i_hbm.at[pl.ds(off, _CHUNK)], idx_v)
            pltpu.async_copy(t_hbm.at[idx_v], rows_v, sem).wait()
            pltpu.sync_copy(rows_v, o_hbm.at[pl.ds(off, _CHUNK)])

    return k(table, idx_flat)


def _tc_body(g_ref, opts_ref, w1_ref, b1_ref, w2_ref, b2_ref,
             w3_ref, b3_ref, wm_ref, bias_ref, esp_ref, out_ref):
    K = g_ref.shape[0]
    NB = g_ref.shape[1]
    KC = w3_ref.shape[1]
    C = wm_ref.shape[0] // KC
    opts = opts_ref[...]
    g2 = g_ref[...].reshape(K * NB, 128)
    rel = g2[:, C:C + 16] - jnp.tile(opts, (K, 1))
    h1 = jax.nn.relu(jnp.dot(rel, w1_ref[...],
                             preferred_element_type=jnp.float32) + b1_ref[...])
    h2 = jax.nn.relu(jnp.dot(h1, w2_ref[...],
                             preferred_element_type=jnp.float32) + b2_ref[...])
    dall = jax.nn.relu(jnp.dot(h2, w3_ref[...],
                               preferred_element_type=jnp.float32) + b3_ref[...])
    d3 = dall.reshape(K, NB, KC)
    accs = [None] * (KC // 2)
    for k in range(K):
        dsp = jnp.dot(d3[k], esp_ref[...], preferred_element_type=jnp.float32)
        f = g_ref[k][:, :C]
        f2 = jnp.concatenate([f, f], axis=1)
        for j2 in range(KC // 2):
            term = f2 * dsp[:, j2 * 2 * C:(j2 + 1) * 2 * C]
            accs[j2] = term if k == 0 else accs[j2] + term
    cat = jnp.concatenate(accs, axis=1)
    out = jnp.dot(cat, wm_ref[...], preferred_element_type=jnp.float32)
    out_ref[...] = out * (1.0 / K) + bias_ref[...]


def kernel(features, input_pts, neighbor_num, output_pts, normalize, indices_,
           weight, bias, centers, l1_W, l1_b, l2_W, l2_b, l3_W, l3_b):
    B, N, C = features.shape
    K = indices_.shape[2]
    D = input_pts.shape[2]
    KC = centers.shape[1]
    H1 = l1_W.shape[0]
    OUT = weight.shape[2]
    NB = 400

    NCH = 10000

    feats2d = features.reshape(N, C)
    pts2d = input_pts.reshape(N, D)
    table = jnp.concatenate(
        [feats2d, pts2d, jnp.zeros((N, 128 - C - D), jnp.float32)], axis=1)
    opts16 = jnp.pad(output_pts.reshape(N, D), ((0, 0), (0, 16 - D)))
    idx2t = indices_.reshape(N, K).T.astype(jnp.int32)

    w1eff = l1_W.reshape(H1, D, KC).sum(-1).T
    w1p = jnp.pad(w1eff, ((0, 16 - D), (0, 0)))
    b1eff = (l1_b - (l1_W.reshape(H1, D, KC) * centers[None]).sum((1, 2)))
    wm = weight.transpose(1, 0, 2).reshape(KC * C, OUT)
    esp = jnp.kron(jnp.eye(KC, dtype=jnp.float32),
                   jnp.ones((1, C), jnp.float32))

    tc_call = pl.pallas_call(
        _tc_body,
        grid=(NCH // NB,),
        in_specs=[
            pl.BlockSpec((K, NB, 128), lambda b: (0, b, 0)),
            pl.BlockSpec((NB, 16), lambda b: (b, 0)),
            pl.BlockSpec((16, H1), lambda b: (0, 0)),
            pl.BlockSpec((1, H1), lambda b: (0, 0)),
            pl.BlockSpec((H1, KC), lambda b: (0, 0)),
            pl.BlockSpec((1, KC), lambda b: (0, 0)),
            pl.BlockSpec((KC, KC), lambda b: (0, 0)),
            pl.BlockSpec((1, KC), lambda b: (0, 0)),
            pl.BlockSpec((KC * C, OUT), lambda b: (0, 0)),
            pl.BlockSpec((1, OUT), lambda b: (0, 0)),
            pl.BlockSpec((KC, KC * C), lambda b: (0, 0)),
        ],
        out_specs=pl.BlockSpec((NB, OUT), lambda b: (b, 0)),
        out_shape=jax.ShapeDtypeStruct((NCH, OUT), jnp.float32),
    )

    outs = []
    for c in range(N // NCH):
        idx_c = lax.slice(idx2t, (0, c * NCH), (K, (c + 1) * NCH)).reshape(-1)
        g3 = _sc_gather_call(table, idx_c).reshape(K, NCH, 128)
        outs.append(tc_call(
            g3, lax.slice(opts16, (c * NCH, 0), ((c + 1) * NCH, 16)),
            w1p, b1eff.reshape(1, H1), l2_W.T, l2_b.reshape(1, KC),
            l3_W.T, l3_b.reshape(1, KC), wm, bias.reshape(1, OUT), esp))
    out = jnp.concatenate(outs, axis=0)

    return (out.reshape(B, N, OUT), output_pts)


# speedup vs baseline: 5.3195x; 1.0238x over previous
import functools

import jax
import jax.numpy as jnp
from jax import lax
from jax.experimental import pallas as pl
from jax.experimental.pallas import tpu as pltpu
from jax.experimental.pallas import tpu_sc as plsc

_NC, _NS = 2, 16
_CHUNK = 1000


def _sc_gather_call(table, idx_flat):
    R = idx_flat.shape[0]
    W = table.shape[1]
    nw = _NC * _NS
    b_per_w = R // nw
    n_ch = b_per_w // _CHUNK
    mesh = plsc.VectorSubcoreMesh(core_axis_name="c", subcore_axis_name="s")

    @functools.partial(
        pl.kernel,
        mesh=mesh,
        out_type=jax.ShapeDtypeStruct((R, W), jnp.float32),
        scratch_types=[
            pltpu.VMEM((_CHUNK,), jnp.int32),
            pltpu.VMEM((_CHUNK, W), jnp.float32),
            pltpu.SemaphoreType.DMA,
        ],
    )
    def k(t_hbm, i_hbm, o_hbm, idx_v, rows_v, sem):
        wid = lax.axis_index("s") * _NC + lax.axis_index("c")
        base = wid * b_per_w

        @pl.loop(0, n_ch)
        def _(c):
            off = base + c * _CHUNK
            pltpu.sync_copy(i_hbm.at[pl.ds(off, _CHUNK)], idx_v)
            pltpu.async_copy(t_hbm.at[idx_v], rows_v, sem).wait()
            pltpu.sync_copy(rows_v, o_hbm.at[pl.ds(off, _CHUNK)])

    return k(table, idx_flat)


def _tc_body(g_ref, opts_ref, w1_ref, b1_ref, w2_ref, b2_ref,
             w3_ref, b3_ref, wm_ref, bias_ref, esp_ref, out_ref):
    K = g_ref.shape[0]
    NB = g_ref.shape[1]
    KC = w3_ref.shape[1]
    C = wm_ref.shape[0] // KC
    opts = opts_ref[...]
    g2 = g_ref[...].reshape(K * NB, 128)
    rel = g2[:, C:C + 16] - jnp.tile(opts, (K, 1))
    h1 = jax.nn.relu(jnp.dot(rel, w1_ref[...],
                             preferred_element_type=jnp.float32) + b1_ref[...])
    h2 = jax.nn.relu(jnp.dot(h1, w2_ref[...],
                             preferred_element_type=jnp.float32) + b2_ref[...])
    dall = jax.nn.relu(jnp.dot(h2, w3_ref[...],
                               preferred_element_type=jnp.float32) + b3_ref[...])
    d3 = dall.reshape(K, NB, KC)
    accs = [None] * (KC // 2)
    for k in range(K):
        dsp = jnp.dot(d3[k], esp_ref[...], preferred_element_type=jnp.float32)
        f = g_ref[k][:, :C]
        f2 = jnp.concatenate([f, f], axis=1)
        for j2 in range(KC // 2):
            term = f2 * dsp[:, j2 * 2 * C:(j2 + 1) * 2 * C]
            accs[j2] = term if k == 0 else accs[j2] + term
    cat = jnp.concatenate(accs, axis=1)
    out = jnp.dot(cat, wm_ref[...], preferred_element_type=jnp.float32)
    out_ref[...] = out * (1.0 / K) + bias_ref[...]


def kernel(features, input_pts, neighbor_num, output_pts, normalize, indices_,
           weight, bias, centers, l1_W, l1_b, l2_W, l2_b, l3_W, l3_b):
    B, N, C = features.shape
    K = indices_.shape[2]
    D = input_pts.shape[2]
    KC = centers.shape[1]
    H1 = l1_W.shape[0]
    OUT = weight.shape[2]
    NB = 1000

    NCH = 10000

    feats2d = features.reshape(N, C)
    pts2d = input_pts.reshape(N, D)
    table = jnp.concatenate(
        [feats2d, pts2d, jnp.zeros((N, 128 - C - D), jnp.float32)], axis=1)
    opts16 = jnp.pad(output_pts.reshape(N, D), ((0, 0), (0, 16 - D)))
    idx2t = indices_.reshape(N, K).T.astype(jnp.int32)

    w1eff = l1_W.reshape(H1, D, KC).sum(-1).T
    w1p = jnp.pad(w1eff, ((0, 16 - D), (0, 0)))
    b1eff = (l1_b - (l1_W.reshape(H1, D, KC) * centers[None]).sum((1, 2)))
    wm = weight.transpose(1, 0, 2).reshape(KC * C, OUT)
    esp = jnp.kron(jnp.eye(KC, dtype=jnp.float32),
                   jnp.ones((1, C), jnp.float32))

    tc_call = pl.pallas_call(
        _tc_body,
        grid=(NCH // NB,),
        in_specs=[
            pl.BlockSpec((K, NB, 128), lambda b: (0, b, 0)),
            pl.BlockSpec((NB, 16), lambda b: (b, 0)),
            pl.BlockSpec((16, H1), lambda b: (0, 0)),
            pl.BlockSpec((1, H1), lambda b: (0, 0)),
            pl.BlockSpec((H1, KC), lambda b: (0, 0)),
            pl.BlockSpec((1, KC), lambda b: (0, 0)),
            pl.BlockSpec((KC, KC), lambda b: (0, 0)),
            pl.BlockSpec((1, KC), lambda b: (0, 0)),
            pl.BlockSpec((KC * C, OUT), lambda b: (0, 0)),
            pl.BlockSpec((1, OUT), lambda b: (0, 0)),
            pl.BlockSpec((KC, KC * C), lambda b: (0, 0)),
        ],
        out_specs=pl.BlockSpec((NB, OUT), lambda b: (b, 0)),
        out_shape=jax.ShapeDtypeStruct((NCH, OUT), jnp.float32),
    )

    outs = []
    for c in range(N // NCH):
        idx_c = lax.slice(idx2t, (0, c * NCH), (K, (c + 1) * NCH)).reshape(-1)
        g3 = _sc_gather_call(table, idx_c).reshape(K, NCH, 128)
        outs.append(tc_call(
            g3, lax.slice(opts16, (c * NCH, 0), ((c + 1) * NCH, 16)),
            w1p, b1eff.reshape(1, H1), l2_W.T, l2_b.reshape(1, KC),
            l3_W.T, l3_b.reshape(1, KC), wm, bias.reshape(1, OUT), esp))
    out = jnp.concatenate(outs, axis=0)

    return (out.reshape(B, N, OUT), output_pts)


# speedup vs baseline: 5.4459x; 1.0238x over previous
import functools

import jax
import jax.numpy as jnp
from jax import lax
from jax.experimental import pallas as pl
from jax.experimental.pallas import tpu as pltpu
from jax.experimental.pallas import tpu_sc as plsc

_NC, _NS = 2, 16
_CHUNK = 1000


def _sc_gather_call(table, idx_flat):
    R = idx_flat.shape[0]
    W = table.shape[1]
    nw = _NC * _NS
    b_per_w = R // nw
    n_ch = b_per_w // _CHUNK
    mesh = plsc.VectorSubcoreMesh(core_axis_name="c", subcore_axis_name="s")

    @functools.partial(
        pl.kernel,
        mesh=mesh,
        out_type=jax.ShapeDtypeStruct((R, W), jnp.float32),
        scratch_types=[
            pltpu.VMEM((_CHUNK,), jnp.int32),
            pltpu.VMEM((_CHUNK, W), jnp.float32),
            pltpu.SemaphoreType.DMA,
        ],
    )
    def k(t_hbm, i_hbm, o_hbm, idx_v, rows_v, sem):
        wid = lax.axis_index("s") * _NC + lax.axis_index("c")
        base = wid * b_per_w

        @pl.loop(0, n_ch)
        def _(c):
            off = base + c * _CHUNK
            pltpu.sync_copy(i_hbm.at[pl.ds(off, _CHUNK)], idx_v)
            pltpu.async_copy(t_hbm.at[idx_v], rows_v, sem).wait()
            pltpu.sync_copy(rows_v, o_hbm.at[pl.ds(off, _CHUNK)])

    return k(table, idx_flat)


def _tc_body(g_ref, opts_ref, w1_ref, b1_ref, w2_ref, b2_ref,
             w3_ref, b3_ref, wm_ref, bias_ref, esp_ref, out_ref):
    K = g_ref.shape[0]
    NB = g_ref.shape[1]
    KC = w3_ref.shape[1]
    C = wm_ref.shape[0] // KC
    opts = opts_ref[...]
    g2 = g_ref[...].reshape(K * NB, 128)
    rel = g2[:, C:C + 16] - jnp.tile(opts, (K, 1))
    h1 = jax.nn.relu(jnp.dot(rel, w1_ref[...],
                             preferred_element_type=jnp.float32) + b1_ref[...])
    h2 = jax.nn.relu(jnp.dot(h1, w2_ref[...],
                             preferred_element_type=jnp.float32) + b2_ref[...])
    dall = jax.nn.relu(jnp.dot(h2, w3_ref[...],
                               preferred_element_type=jnp.float32) + b3_ref[...])
    d3 = dall.reshape(K, NB, KC)
    accs = [None] * (KC // 2)
    for k in range(K):
        dsp = jnp.dot(d3[k], esp_ref[...], preferred_element_type=jnp.float32)
        f = g_ref[k][:, :C]
        f2 = jnp.concatenate([f, f], axis=1)
        for j2 in range(KC // 2):
            term = f2 * dsp[:, j2 * 2 * C:(j2 + 1) * 2 * C]
            accs[j2] = term if k == 0 else accs[j2] + term
    cat = jnp.concatenate(accs, axis=1)
    out = jnp.dot(cat, wm_ref[...], preferred_element_type=jnp.float32)
    out_ref[...] = out * (1.0 / K) + bias_ref[...]


def kernel(features, input_pts, neighbor_num, output_pts, normalize, indices_,
           weight, bias, centers, l1_W, l1_b, l2_W, l2_b, l3_W, l3_b):
    B, N, C = features.shape
    K = indices_.shape[2]
    D = input_pts.shape[2]
    KC = centers.shape[1]
    H1 = l1_W.shape[0]
    OUT = weight.shape[2]
    NB = 1000

    chunks = (2000, 4000, 8000, 16000, 20000)

    feats2d = features.reshape(N, C)
    pts2d = input_pts.reshape(N, D)
    table = jnp.concatenate(
        [feats2d, pts2d, jnp.zeros((N, 128 - C - D), jnp.float32)], axis=1)
    opts16 = jnp.pad(output_pts.reshape(N, D), ((0, 0), (0, 16 - D)))
    idx2t = indices_.reshape(N, K).T.astype(jnp.int32)

    w1eff = l1_W.reshape(H1, D, KC).sum(-1).T
    w1p = jnp.pad(w1eff, ((0, 16 - D), (0, 0)))
    b1eff = (l1_b - (l1_W.reshape(H1, D, KC) * centers[None]).sum((1, 2)))
    wm = weight.transpose(1, 0, 2).reshape(KC * C, OUT)
    esp = jnp.kron(jnp.eye(KC, dtype=jnp.float32),
                   jnp.ones((1, C), jnp.float32))

    def make_tc_call(nch):
        return pl.pallas_call(
            _tc_body,
            grid=(nch // NB,),
            in_specs=[
                pl.BlockSpec((K, NB, 128), lambda b: (0, b, 0)),
                pl.BlockSpec((NB, 16), lambda b: (b, 0)),
                pl.BlockSpec((16, H1), lambda b: (0, 0)),
                pl.BlockSpec((1, H1), lambda b: (0, 0)),
                pl.BlockSpec((H1, KC), lambda b: (0, 0)),
                pl.BlockSpec((1, KC), lambda b: (0, 0)),
                pl.BlockSpec((KC, KC), lambda b: (0, 0)),
                pl.BlockSpec((1, KC), lambda b: (0, 0)),
                pl.BlockSpec((KC * C, OUT), lambda b: (0, 0)),
                pl.BlockSpec((1, OUT), lambda b: (0, 0)),
                pl.BlockSpec((KC, KC * C), lambda b: (0, 0)),
            ],
            out_specs=pl.BlockSpec((NB, OUT), lambda b: (b, 0)),
            out_shape=jax.ShapeDtypeStruct((nch, OUT), jnp.float32),
        )

    outs = []
    n0 = 0
    for nch in chunks:
        idx_c = lax.slice(idx2t, (0, n0), (K, n0 + nch)).reshape(-1)
        g3 = _sc_gather_call(table, idx_c).reshape(K, nch, 128)
        outs.append(make_tc_call(nch)(
            g3, lax.slice(opts16, (n0, 0), (n0 + nch, 16)),
            w1p, b1eff.reshape(1, H1), l2_W.T, l2_b.reshape(1, KC),
            l3_W.T, l3_b.reshape(1, KC), wm, bias.reshape(1, OUT), esp))
        n0 += nch
    out = jnp.concatenate(outs, axis=0)

    return (out.reshape(B, N, OUT), output_pts)
